# baseline (speedup 1.0000x reference)
import functools

import jax
import jax.numpy as jnp
from jax import lax
from jax.experimental import pallas as pl
from jax.experimental.pallas import tpu as pltpu
from jax.experimental.pallas import tpu_sc as plsc

_NC = 2
_NS = 16
_NW = _NC * _NS


def _tc_body(a_ref, q_ref, idx_ref, *, pblk, m):
    i = pl.program_id(0)
    sims = [jax.lax.dot_general(
        q_ref[p], a_ref[:, p, :], (((1,), (1,)), ((), ())),
        preferred_element_type=jnp.float32) for p in range(pblk)]
    for p in range(pblk):
        idx_ref[p, :] = jnp.argmax(sims[p], axis=0) + (i * pblk + p) * m


def _sc_body(fidx_hbm, qflat_hbm, out_hbm, pos_v, fidx_v, rows_v, s1, s2,
             *, rows_total, f, b_total, p_total, ch):
    wid = lax.axis_index("s") * _NC + lax.axis_index("c")
    per_w = rows_total // _NW
    nch = per_w // ch
    lane = lax.iota(jnp.int32, 16)
    for c in range(nch):
        r0 = wid * per_w + c * ch
        b = r0 // p_total
        p0 = r0 % p_total
        for j in range(ch // 16):
            pos_v[pl.ds(j * 16, 16)] = (p0 + j * 16 + lane) * b_total + b
        pltpu.async_copy(fidx_hbm.at[pos_v], fidx_v, s1).wait()
        pltpu.async_copy(qflat_hbm.at[fidx_v], rows_v, s2).wait()
        pltpu.sync_copy(rows_v, out_hbm.at[pl.ds(r0, ch)])


@jax.jit
def kernel(patch_features, queue):
    b, p_total, f = patch_features.shape
    _, m, _ = queue.shape
    pblk = 32
    fidx = pl.pallas_call(
        functools.partial(_tc_body, pblk=pblk, m=m),
        grid=(p_total // pblk,),
        in_specs=[
            pl.BlockSpec((b, pblk, f), lambda i: (0, i, 0)),
            pl.BlockSpec((pblk, m, f), lambda i: (i, 0, 0)),
        ],
        out_specs=pl.BlockSpec((pblk, b), lambda i: (i, 0)),
        out_shape=jax.ShapeDtypeStruct((p_total, b), jnp.int32),
        compiler_params=pltpu.CompilerParams(
            dimension_semantics=("arbitrary",)),
    )(patch_features, queue)

    rows_total = b * p_total
    ch = 96
    mesh = plsc.VectorSubcoreMesh(core_axis_name="c", subcore_axis_name="s")
    sc_gather = functools.partial(
        pl.kernel,
        out_type=jax.ShapeDtypeStruct((rows_total, f), jnp.float32),
        mesh=mesh,
        scratch_types=[
            pltpu.VMEM((ch,), jnp.int32),
            pltpu.VMEM((ch,), jnp.int32),
            pltpu.VMEM((ch, f), jnp.float32),
            pltpu.SemaphoreType.DMA,
            pltpu.SemaphoreType.DMA,
        ],
    )(functools.partial(
        _sc_body, rows_total=rows_total, f=f, b_total=b, p_total=p_total,
        ch=ch))
    out_flat = sc_gather(fidx.reshape(rows_total), queue.reshape(p_total * m, f))
    return out_flat.reshape(b, p_total, f)


# speedup vs baseline: 1.4499x; 1.4499x over previous
import functools

import jax
import jax.numpy as jnp
from jax.experimental import pallas as pl
from jax.experimental.pallas import tpu as pltpu


def _body(a_ref, q_ref, o_ref, *, pblk):
    m = q_ref.shape[1]
    iota = jax.lax.broadcasted_iota(jnp.int32, (1, m), 1)
    sims = [jax.lax.dot_general(
        a_ref[:, p, :], q_ref[p], (((1,), (1,)), ((), ())),
        preferred_element_type=jnp.float32) for p in range(pblk)]
    onehots = [(jnp.argmax(sim, axis=1)[:, None] == iota).astype(jnp.float32)
               for sim in sims]
    for p in range(pblk):
        o_ref[:, p, :] = jax.lax.dot_general(
            onehots[p], q_ref[p], (((1,), (0,)), ((), ())),
            preferred_element_type=jnp.float32)


@jax.jit
def kernel(patch_features, queue):
    b, p_total, f = patch_features.shape
    _, m, _ = queue.shape
    pblk = 32
    grid = (p_total // pblk,)
    return pl.pallas_call(
        functools.partial(_body, pblk=pblk),
        grid=grid,
        in_specs=[
            pl.BlockSpec((b, pblk, f), lambda i: (0, i, 0)),
            pl.BlockSpec((pblk, m, f), lambda i: (i, 0, 0)),
        ],
        out_specs=pl.BlockSpec((b, pblk, f), lambda i: (0, i, 0)),
        out_shape=jax.ShapeDtypeStruct((b, p_total, f), jnp.float32),
        compiler_params=pltpu.CompilerParams(
            dimension_semantics=("parallel",)),
    )(patch_features, queue)
